# initial kernel scaffold (unmeasured)
import jax
import jax.numpy as jnp
from jax import lax
from jax.experimental import pallas as pl
from jax.experimental.pallas import tpu as pltpu


def kernel(
    x,
):
    def body(*refs):
        pass

    out_shape = jax.ShapeDtypeStruct(..., jnp.float32)
    return pl.pallas_call(body, out_shape=out_shape)(...)


# baseline (device time: 101402 ns/iter reference)
import jax
import jax.numpy as jnp
from jax import lax
from jax.experimental import pallas as pl
from jax.experimental.pallas import tpu as pltpu

N_DEV = 8
M = 1024
N_CHUNK = 512


def kernel(x):
    x = x.reshape(M, N_DEV * N_CHUNK)

    def body(x_ref, out_ref, comm_ref, acc_ref, send_sem, recv_sems):
        my = lax.axis_index("i")
        left = lax.rem(my + N_DEV - 1, N_DEV)
        right = lax.rem(my + 1, N_DEV)

        barrier_sem = pltpu.get_barrier_semaphore()
        for nbr in (left, right):
            pl.semaphore_signal(
                barrier_sem, inc=1,
                device_id=(nbr,), device_id_type=pl.DeviceIdType.MESH,
            )
        pl.semaphore_wait(barrier_sem, 2)

        c0 = lax.rem(my + N_DEV - 1, N_DEV)
        acc_ref[...] = x_ref[:, pl.ds(c0 * N_CHUNK, N_CHUNK)].astype(
            jnp.bfloat16
        )

        for s in range(N_DEV - 1):
            rdma = pltpu.make_async_remote_copy(
                src_ref=acc_ref,
                dst_ref=comm_ref.at[s],
                send_sem=send_sem,
                recv_sem=recv_sems.at[s],
                device_id=(right,),
                device_id_type=pl.DeviceIdType.MESH,
            )
            rdma.start()
            rdma.wait()
            c = lax.rem(my + 2 * N_DEV - 2 - s, N_DEV)
            acc_ref[...] = comm_ref[s] + x_ref[
                :, pl.ds(c * N_CHUNK, N_CHUNK)
            ].astype(jnp.bfloat16)

        out_ref[...] = acc_ref[...]

    return pl.pallas_call(
        body,
        out_shape=jax.ShapeDtypeStruct((M, N_CHUNK), jnp.bfloat16),
        in_specs=[pl.BlockSpec(memory_space=pltpu.VMEM)],
        out_specs=pl.BlockSpec(memory_space=pltpu.VMEM),
        scratch_shapes=[
            pltpu.VMEM((N_DEV - 1, M, N_CHUNK), jnp.bfloat16),
            pltpu.VMEM((M, N_CHUNK), jnp.bfloat16),
            pltpu.SemaphoreType.DMA,
            pltpu.SemaphoreType.DMA((N_DEV - 1,)),
        ],
        compiler_params=pltpu.CompilerParams(collective_id=0),
    )(x)


# device time: 66309 ns/iter; 1.5292x vs baseline; 1.5292x over previous
import jax
import jax.numpy as jnp
from jax import lax
from jax.experimental import pallas as pl
from jax.experimental.pallas import tpu as pltpu

N_DEV = 8
M = 1024
M_HALF = M // 2
N_CHUNK = 512


def kernel(x):
    x = x.reshape(M, N_DEV * N_CHUNK)

    def body(
        x_ref,
        out_ref,
        comm_a,
        comm_b,
        acc_a,
        acc_b,
        send_sem_a,
        send_sem_b,
        recv_sems_a,
        recv_sems_b,
    ):
        my = lax.axis_index("i")
        left = lax.rem(my + N_DEV - 1, N_DEV)
        right = lax.rem(my + 1, N_DEV)

        barrier_sem = pltpu.get_barrier_semaphore()
        for nbr in (left, right):
            pl.semaphore_signal(
                barrier_sem, inc=1,
                device_id=(nbr,), device_id_type=pl.DeviceIdType.MESH,
            )
        pl.semaphore_wait(barrier_sem, 2)

        ca0 = lax.rem(my + N_DEV - 1, N_DEV)
        cb0 = lax.rem(my + 1, N_DEV)
        acc_a[...] = x_ref[:M_HALF, pl.ds(ca0 * N_CHUNK, N_CHUNK)].astype(
            jnp.bfloat16
        )
        acc_b[...] = x_ref[M_HALF:, pl.ds(cb0 * N_CHUNK, N_CHUNK)].astype(
            jnp.bfloat16
        )

        for s in range(N_DEV - 1):
            rdma_a = pltpu.make_async_remote_copy(
                src_ref=acc_a,
                dst_ref=comm_a.at[s],
                send_sem=send_sem_a,
                recv_sem=recv_sems_a.at[s],
                device_id=(right,),
                device_id_type=pl.DeviceIdType.MESH,
            )
            rdma_b = pltpu.make_async_remote_copy(
                src_ref=acc_b,
                dst_ref=comm_b.at[s],
                send_sem=send_sem_b,
                recv_sem=recv_sems_b.at[s],
                device_id=(left,),
                device_id_type=pl.DeviceIdType.MESH,
            )
            rdma_a.start()
            rdma_b.start()
            rdma_a.wait()
            rdma_b.wait()
            ca = lax.rem(my + 2 * N_DEV - 2 - s, N_DEV)
            cb = lax.rem(my + 2 + s, N_DEV)
            acc_a[...] = comm_a[s] + x_ref[
                :M_HALF, pl.ds(ca * N_CHUNK, N_CHUNK)
            ].astype(jnp.bfloat16)
            acc_b[...] = comm_b[s] + x_ref[
                M_HALF:, pl.ds(cb * N_CHUNK, N_CHUNK)
            ].astype(jnp.bfloat16)

        out_ref[:M_HALF, :] = acc_a[...]
        out_ref[M_HALF:, :] = acc_b[...]

    return pl.pallas_call(
        body,
        out_shape=jax.ShapeDtypeStruct((M, N_CHUNK), jnp.bfloat16),
        in_specs=[pl.BlockSpec(memory_space=pltpu.VMEM)],
        out_specs=pl.BlockSpec(memory_space=pltpu.VMEM),
        scratch_shapes=[
            pltpu.VMEM((N_DEV - 1, M_HALF, N_CHUNK), jnp.bfloat16),
            pltpu.VMEM((N_DEV - 1, M_HALF, N_CHUNK), jnp.bfloat16),
            pltpu.VMEM((M_HALF, N_CHUNK), jnp.bfloat16),
            pltpu.VMEM((M_HALF, N_CHUNK), jnp.bfloat16),
            pltpu.SemaphoreType.DMA,
            pltpu.SemaphoreType.DMA,
            pltpu.SemaphoreType.DMA((N_DEV - 1,)),
            pltpu.SemaphoreType.DMA((N_DEV - 1,)),
        ],
        compiler_params=pltpu.CompilerParams(collective_id=0),
    )(x)


# device time: 52808 ns/iter; 1.9202x vs baseline; 1.2557x over previous
import jax
import jax.numpy as jnp
from jax import lax
from jax.experimental import pallas as pl
from jax.experimental.pallas import tpu as pltpu

N_DEV = 8
N_HOP = N_DEV - 1
M = 1024
M_HALF = M // 2
N_CHUNK = 512
K_SUB = 4
N_SUB = N_CHUNK // K_SUB


def kernel(x):
    x = x.reshape(M, N_DEV * N_CHUNK)

    def body(
        x_ref,
        out_ref,
        comm_a,
        comm_b,
        acc_a,
        acc_b,
        send_sems_a,
        send_sems_b,
        recv_sems_a,
        recv_sems_b,
    ):
        my = lax.axis_index("i")
        left = lax.rem(my + N_DEV - 1, N_DEV)
        right = lax.rem(my + 1, N_DEV)

        barrier_sem = pltpu.get_barrier_semaphore()
        for nbr in (left, right):
            pl.semaphore_signal(
                barrier_sem, inc=1,
                device_id=(nbr,), device_id_type=pl.DeviceIdType.MESH,
            )
        pl.semaphore_wait(barrier_sem, 2)

        def col(c, k):
            return pl.ds(c * N_CHUNK + k * N_SUB, N_SUB)

        def send(dir_tag, h, k):
            acc, comm, ssems, rsems, tgt = (
                (acc_a, comm_a, send_sems_a, recv_sems_a, right)
                if dir_tag == 0
                else (acc_b, comm_b, send_sems_b, recv_sems_b, left)
            )
            return pltpu.make_async_remote_copy(
                src_ref=acc.at[k],
                dst_ref=comm.at[h, k],
                send_sem=ssems.at[k],
                recv_sem=rsems.at[h, k],
                device_id=(tgt,),
                device_id_type=pl.DeviceIdType.MESH,
            )

        ca0 = lax.rem(my + N_DEV - 1, N_DEV)
        cb0 = lax.rem(my + 1, N_DEV)
        for k in range(K_SUB):
            acc_a[k] = x_ref[:M_HALF, col(ca0, k)].astype(jnp.bfloat16)
            send(0, 0, k).start()
            acc_b[k] = x_ref[M_HALF:, col(cb0, k)].astype(jnp.bfloat16)
            send(1, 0, k).start()

        for h in range(N_HOP):
            ca = lax.rem(my + 2 * N_DEV - 2 - h, N_DEV)
            cb = lax.rem(my + 2 + h, N_DEV)
            for k in range(K_SUB):
                send(0, h, k).wait_recv()
                if h < N_HOP - 1:
                    send(0, h, k).wait_send()
                    acc_a[k] = comm_a[h, k] + x_ref[
                        :M_HALF, col(ca, k)
                    ].astype(jnp.bfloat16)
                    send(0, h + 1, k).start()
                else:
                    out_ref[:M_HALF, pl.ds(k * N_SUB, N_SUB)] = (
                        comm_a[h, k]
                        + x_ref[:M_HALF, col(ca, k)].astype(jnp.bfloat16)
                    )
                send(1, h, k).wait_recv()
                if h < N_HOP - 1:
                    send(1, h, k).wait_send()
                    acc_b[k] = comm_b[h, k] + x_ref[
                        M_HALF:, col(cb, k)
                    ].astype(jnp.bfloat16)
                    send(1, h + 1, k).start()
                else:
                    out_ref[M_HALF:, pl.ds(k * N_SUB, N_SUB)] = (
                        comm_b[h, k]
                        + x_ref[M_HALF:, col(cb, k)].astype(jnp.bfloat16)
                    )

        for k in range(K_SUB):
            send(0, N_HOP - 1, k).wait_send()
            send(1, N_HOP - 1, k).wait_send()

    return pl.pallas_call(
        body,
        out_shape=jax.ShapeDtypeStruct((M, N_CHUNK), jnp.bfloat16),
        in_specs=[pl.BlockSpec(memory_space=pltpu.VMEM)],
        out_specs=pl.BlockSpec(memory_space=pltpu.VMEM),
        scratch_shapes=[
            pltpu.VMEM((N_HOP, K_SUB, M_HALF, N_SUB), jnp.bfloat16),
            pltpu.VMEM((N_HOP, K_SUB, M_HALF, N_SUB), jnp.bfloat16),
            pltpu.VMEM((K_SUB, M_HALF, N_SUB), jnp.bfloat16),
            pltpu.VMEM((K_SUB, M_HALF, N_SUB), jnp.bfloat16),
            pltpu.SemaphoreType.DMA((K_SUB,)),
            pltpu.SemaphoreType.DMA((K_SUB,)),
            pltpu.SemaphoreType.DMA((N_HOP, K_SUB)),
            pltpu.SemaphoreType.DMA((N_HOP, K_SUB)),
        ],
        compiler_params=pltpu.CompilerParams(collective_id=0),
    )(x)
